# Initial kernel scaffold; baseline (speedup 1.0000x reference)
import math

import jax
import jax.numpy as jnp
from jax.experimental import pallas as pl
from jax.experimental.pallas import tpu as pltpu


_BLOCK_NAMES = (
    "conv0_0", "conv1_0", "conv2_0", "conv3_0", "conv4_0",
    "conv0_1", "conv1_1", "conv2_1", "conv3_1",
    "conv0_2", "conv1_2", "conv2_2",
    "conv0_3", "conv1_3",
    "conv0_4",
)


def _lerp_coeffs(size_in):
    size_out = 2 * size_in
    if size_in == 1:
        return tuple((0, 0, 0.0) for _ in range(size_out))
    coeffs = []
    for o in range(size_out):
        src = o * (size_in - 1) / (size_out - 1)
        lo = min(int(math.floor(src)), size_in - 2)
        coeffs.append((lo, lo + 1, float(src - lo)))
    return tuple(coeffs)


def _upsample2x(v):
    n, h, w, c = v.shape
    x = v.astype(jnp.float32)

    def lerp_axis2(u, coeffs):
        pieces = []
        for lo, hi, f in coeffs:
            if f == 0.0:
                pieces.append(u[:, :, lo, :])
            elif f == 1.0:
                pieces.append(u[:, :, hi, :])
            else:
                pieces.append((1.0 - f) * u[:, :, lo, :] + f * u[:, :, hi, :])
        return jnp.stack(pieces, axis=1)

    y = lerp_axis2(x, _lerp_coeffs(w))
    z = lerp_axis2(y, _lerp_coeffs(h))
    return z.astype(jnp.bfloat16)


def _maxpool2x2(v):
    n, h, w, c = v.shape
    vr = v.reshape(n, h // 2, 2, w, c)
    m = jnp.maximum(vr[:, :, 0], vr[:, :, 1])
    mr = m.reshape(n, h // 2, w // 2, 2 * c)
    return jnp.maximum(mr[..., :c], mr[..., c:])


def _conv3x3_bn_relu(v, w_ref, t_ref):
    n, h, w, c = v.shape
    cout = w_ref.shape[-1]
    zh = jnp.zeros((n, 1, w, c), v.dtype)
    p = jnp.concatenate([zh, v, zh], axis=1)
    zw = jnp.zeros((n, h + 2, 1, c), v.dtype)
    p = jnp.concatenate([zw, p, zw], axis=2)
    cols = []
    for dy in range(3):
        for dx in range(3):
            cols.append(p[:, dy:dy + h, dx:dx + w, :].reshape(n * h * w, c))
    patches = jnp.concatenate(cols, axis=-1)
    acc = jnp.dot(patches, w_ref[...],
                  preferred_element_type=jnp.float32)
    y = jnp.maximum(acc + t_ref[...], 0.0)
    return y.astype(jnp.bfloat16).reshape(n, h, w, cout)


def _unet_kernel(*refs):
    x_ref = refs[0]
    o_ref = refs[-1]
    wrefs = refs[1:-1]
    blk = {name: wrefs[4 * i:4 * i + 4] for i, name in enumerate(_BLOCK_NAMES)}
    final_w, final_b = wrefs[60], wrefs[61]

    def block(inputs, name):
        w1, t1, w2, t2 = blk[name]
        v = inputs[0] if len(inputs) == 1 else jnp.concatenate(inputs, axis=-1)
        y1 = _conv3x3_bn_relu(v, w1, t1)
        return _conv3x3_bn_relu(y1, w2, t2)

    up = _upsample2x
    pool = _maxpool2x2

    x = x_ref[...]
    x0_0 = block([x], "conv0_0")
    x1_0 = block([pool(x0_0)], "conv1_0")
    x0_1 = block([x0_0, up(x1_0)], "conv0_1")

    x2_0 = block([pool(x1_0)], "conv2_0")
    x1_1 = block([x1_0, up(x2_0)], "conv1_1")
    x0_2 = block([x0_0, x0_1, up(x1_1)], "conv0_2")

    x3_0 = block([pool(x2_0)], "conv3_0")
    x2_1 = block([x2_0, up(x3_0)], "conv2_1")
    x1_2 = block([x1_0, x1_1, up(x2_1)], "conv1_2")
    x0_3 = block([x0_0, x0_1, x0_2, up(x1_2)], "conv0_3")

    x4_0 = block([pool(x3_0)], "conv4_0")
    x3_1 = block([x3_0, up(x4_0)], "conv3_1")
    x2_2 = block([x2_0, x2_1, up(x3_1)], "conv2_2")
    x1_3 = block([x1_0, x1_1, x1_2, up(x2_2)], "conv1_3")
    x0_4 = block([x0_0, x0_1, x0_2, x0_3, up(x1_3)], "conv0_4")

    n, h, w, c = x0_4.shape
    k = final_w.shape[-1]
    y = jnp.dot(x0_4.reshape(n * h * w, c), final_w[...],
                preferred_element_type=jnp.float32) + final_b[...]
    o_ref[...] = y.reshape(n, h, w, k)


def _full_spec(shape):
    ndim = len(shape)
    return pl.BlockSpec(tuple(shape), lambda i, _n=ndim: (0,) * _n)


@jax.jit
def _forward(x, *weights):
    n, hh, ww = x.shape[0], x.shape[2], x.shape[3]
    xh = jnp.transpose(x, (0, 2, 3, 1)).astype(jnp.bfloat16)
    num_classes = weights[-2].shape[-1]

    nsplit = 2 if n % 2 == 0 else 1
    nb = n // nsplit

    out = pl.pallas_call(
        _unet_kernel,
        out_shape=jax.ShapeDtypeStruct((n, hh, ww, num_classes), jnp.float32),
        grid=(nsplit,),
        in_specs=[pl.BlockSpec((nb, hh, ww, xh.shape[-1]),
                               lambda i: (i, 0, 0, 0))]
                 + [_full_spec(wt.shape) for wt in weights],
        out_specs=pl.BlockSpec((nb, hh, ww, num_classes),
                               lambda i: (i, 0, 0, 0)),
        compiler_params=pltpu.CompilerParams(
            dimension_semantics=(pltpu.CORE_PARALLEL,),
            vmem_limit_bytes=100 * 1024 * 1024,
        ),
    )(xh, *weights)
    return jnp.transpose(out, (0, 3, 1, 2))


def kernel(x, conv0_0_w1, conv0_0_t1, conv0_0_w2, conv0_0_t2, conv1_0_w1, conv1_0_t1, conv1_0_w2, conv1_0_t2, conv2_0_w1, conv2_0_t1, conv2_0_w2, conv2_0_t2, conv3_0_w1, conv3_0_t1, conv3_0_w2, conv3_0_t2, conv4_0_w1, conv4_0_t1, conv4_0_w2, conv4_0_t2, conv0_1_w1, conv0_1_t1, conv0_1_w2, conv0_1_t2, conv1_1_w1, conv1_1_t1, conv1_1_w2, conv1_1_t2, conv2_1_w1, conv2_1_t1, conv2_1_w2, conv2_1_t2, conv3_1_w1, conv3_1_t1, conv3_1_w2, conv3_1_t2, conv0_2_w1, conv0_2_t1, conv0_2_w2, conv0_2_t2, conv1_2_w1, conv1_2_t1, conv1_2_w2, conv1_2_t2, conv2_2_w1, conv2_2_t1, conv2_2_w2, conv2_2_t2, conv0_3_w1, conv0_3_t1, conv0_3_w2, conv0_3_t2, conv1_3_w1, conv1_3_t1, conv1_3_w2, conv1_3_t2, conv0_4_w1, conv0_4_t1, conv0_4_w2, conv0_4_t2, final_w, final_b):
    return _forward(
        x,
        conv0_0_w1, conv0_0_t1, conv0_0_w2, conv0_0_t2,
        conv1_0_w1, conv1_0_t1, conv1_0_w2, conv1_0_t2,
        conv2_0_w1, conv2_0_t1, conv2_0_w2, conv2_0_t2,
        conv3_0_w1, conv3_0_t1, conv3_0_w2, conv3_0_t2,
        conv4_0_w1, conv4_0_t1, conv4_0_w2, conv4_0_t2,
        conv0_1_w1, conv0_1_t1, conv0_1_w2, conv0_1_t2,
        conv1_1_w1, conv1_1_t1, conv1_1_w2, conv1_1_t2,
        conv2_1_w1, conv2_1_t1, conv2_1_w2, conv2_1_t2,
        conv3_1_w1, conv3_1_t1, conv3_1_w2, conv3_1_t2,
        conv0_2_w1, conv0_2_t1, conv0_2_w2, conv0_2_t2,
        conv1_2_w1, conv1_2_t1, conv1_2_w2, conv1_2_t2,
        conv2_2_w1, conv2_2_t1, conv2_2_w2, conv2_2_t2,
        conv0_3_w1, conv0_3_t1, conv0_3_w2, conv0_3_t2,
        conv1_3_w1, conv1_3_t1, conv1_3_w2, conv1_3_t2,
        conv0_4_w1, conv0_4_t1, conv0_4_w2, conv0_4_t2,
        final_w, final_b,
    )


# speedup vs baseline: 1.0975x; 1.0975x over previous
import math

import jax
import jax.numpy as jnp
from jax.experimental import pallas as pl
from jax.experimental.pallas import tpu as pltpu


_BLOCK_NAMES = (
    "conv0_0", "conv1_0", "conv2_0", "conv3_0", "conv4_0",
    "conv0_1", "conv1_1", "conv2_1", "conv3_1",
    "conv0_2", "conv1_2", "conv2_2",
    "conv0_3", "conv1_3",
    "conv0_4",
)


def _lerp_coeffs(size_in):
    size_out = 2 * size_in
    if size_in == 1:
        return tuple((0, 0, 0.0) for _ in range(size_out))
    coeffs = []
    for o in range(size_out):
        src = o * (size_in - 1) / (size_out - 1)
        lo = min(int(math.floor(src)), size_in - 2)
        coeffs.append((lo, lo + 1, float(src - lo)))
    return tuple(coeffs)


def _upsample2x(v):
    n, h, w, c = v.shape
    x = v.astype(jnp.float32)

    def lerp_axis2(u, coeffs):
        pieces = []
        for lo, hi, f in coeffs:
            if f == 0.0:
                pieces.append(u[:, :, lo, :])
            elif f == 1.0:
                pieces.append(u[:, :, hi, :])
            else:
                pieces.append((1.0 - f) * u[:, :, lo, :] + f * u[:, :, hi, :])
        return jnp.stack(pieces, axis=1)

    y = lerp_axis2(x, _lerp_coeffs(w))
    z = lerp_axis2(y, _lerp_coeffs(h))
    return z.astype(jnp.bfloat16)


def _maxpool2x2(v):
    n, h, w, c = v.shape
    vr = v.reshape(n, h // 2, 2, w, c)
    m = jnp.maximum(vr[:, :, 0], vr[:, :, 1])
    pieces = [jnp.maximum(m[:, :, 2 * j, :], m[:, :, 2 * j + 1, :])
              for j in range(w // 2)]
    return jnp.stack(pieces, axis=2)


def _conv3x3_bn_relu(v, w_ref, t_ref):
    n, h, w, c = v.shape
    cout = w_ref.shape[-1]
    zh = jnp.zeros((n, 1, w, c), v.dtype)
    p = jnp.concatenate([zh, v, zh], axis=1)
    zw = jnp.zeros((n, h + 2, 1, c), v.dtype)
    p = jnp.concatenate([zw, p, zw], axis=2)
    cols = []
    for dy in range(3):
        for dx in range(3):
            cols.append(p[:, dy:dy + h, dx:dx + w, :].reshape(n * h * w, c))
    patches = jnp.concatenate(cols, axis=-1)
    acc = jnp.dot(patches, w_ref[...],
                  preferred_element_type=jnp.float32)
    y = jnp.maximum(acc + t_ref[...], 0.0)
    return y.astype(jnp.bfloat16).reshape(n, h, w, cout)


def _unet_kernel(*refs):
    x_ref = refs[0]
    o_ref = refs[-1]
    wrefs = refs[1:-1]
    blk = {name: wrefs[4 * i:4 * i + 4] for i, name in enumerate(_BLOCK_NAMES)}
    final_w, final_b = wrefs[60], wrefs[61]

    def block(inputs, name):
        w1, t1, w2, t2 = blk[name]
        v = inputs[0] if len(inputs) == 1 else jnp.concatenate(inputs, axis=-1)
        y1 = _conv3x3_bn_relu(v, w1, t1)
        return _conv3x3_bn_relu(y1, w2, t2)

    up = _upsample2x
    pool = _maxpool2x2

    x = x_ref[...]
    x0_0 = block([x], "conv0_0")
    x1_0 = block([pool(x0_0)], "conv1_0")
    x0_1 = block([x0_0, up(x1_0)], "conv0_1")

    x2_0 = block([pool(x1_0)], "conv2_0")
    x1_1 = block([x1_0, up(x2_0)], "conv1_1")
    x0_2 = block([x0_0, x0_1, up(x1_1)], "conv0_2")

    x3_0 = block([pool(x2_0)], "conv3_0")
    x2_1 = block([x2_0, up(x3_0)], "conv2_1")
    x1_2 = block([x1_0, x1_1, up(x2_1)], "conv1_2")
    x0_3 = block([x0_0, x0_1, x0_2, up(x1_2)], "conv0_3")

    x4_0 = block([pool(x3_0)], "conv4_0")
    x3_1 = block([x3_0, up(x4_0)], "conv3_1")
    x2_2 = block([x2_0, x2_1, up(x3_1)], "conv2_2")
    x1_3 = block([x1_0, x1_1, x1_2, up(x2_2)], "conv1_3")
    x0_4 = block([x0_0, x0_1, x0_2, x0_3, up(x1_3)], "conv0_4")

    n, h, w, c = x0_4.shape
    k = final_w.shape[-1]
    y = jnp.dot(x0_4.reshape(n * h * w, c), final_w[...],
                preferred_element_type=jnp.float32) + final_b[...]
    o_ref[...] = y.reshape(n, h, w, k)


def _full_spec(shape):
    ndim = len(shape)
    return pl.BlockSpec(tuple(shape), lambda i, _n=ndim: (0,) * _n)


@jax.jit
def _forward(x, *weights):
    n, hh, ww = x.shape[0], x.shape[2], x.shape[3]
    xh = jnp.transpose(x, (0, 2, 3, 1)).astype(jnp.bfloat16)
    num_classes = weights[-2].shape[-1]

    out = pl.pallas_call(
        _unet_kernel,
        out_shape=jax.ShapeDtypeStruct((n, hh, ww, num_classes), jnp.float32),
        grid=(1,),
        in_specs=[_full_spec(xh.shape)]
                 + [_full_spec(wt.shape) for wt in weights],
        out_specs=_full_spec((n, hh, ww, num_classes)),
        compiler_params=pltpu.CompilerParams(
            dimension_semantics=("arbitrary",),
            vmem_limit_bytes=100 * 1024 * 1024,
        ),
    )(xh, *weights)
    return jnp.transpose(out, (0, 3, 1, 2))


def kernel(x, conv0_0_w1, conv0_0_t1, conv0_0_w2, conv0_0_t2, conv1_0_w1, conv1_0_t1, conv1_0_w2, conv1_0_t2, conv2_0_w1, conv2_0_t1, conv2_0_w2, conv2_0_t2, conv3_0_w1, conv3_0_t1, conv3_0_w2, conv3_0_t2, conv4_0_w1, conv4_0_t1, conv4_0_w2, conv4_0_t2, conv0_1_w1, conv0_1_t1, conv0_1_w2, conv0_1_t2, conv1_1_w1, conv1_1_t1, conv1_1_w2, conv1_1_t2, conv2_1_w1, conv2_1_t1, conv2_1_w2, conv2_1_t2, conv3_1_w1, conv3_1_t1, conv3_1_w2, conv3_1_t2, conv0_2_w1, conv0_2_t1, conv0_2_w2, conv0_2_t2, conv1_2_w1, conv1_2_t1, conv1_2_w2, conv1_2_t2, conv2_2_w1, conv2_2_t1, conv2_2_w2, conv2_2_t2, conv0_3_w1, conv0_3_t1, conv0_3_w2, conv0_3_t2, conv1_3_w1, conv1_3_t1, conv1_3_w2, conv1_3_t2, conv0_4_w1, conv0_4_t1, conv0_4_w2, conv0_4_t2, final_w, final_b):
    return _forward(
        x,
        conv0_0_w1, conv0_0_t1, conv0_0_w2, conv0_0_t2,
        conv1_0_w1, conv1_0_t1, conv1_0_w2, conv1_0_t2,
        conv2_0_w1, conv2_0_t1, conv2_0_w2, conv2_0_t2,
        conv3_0_w1, conv3_0_t1, conv3_0_w2, conv3_0_t2,
        conv4_0_w1, conv4_0_t1, conv4_0_w2, conv4_0_t2,
        conv0_1_w1, conv0_1_t1, conv0_1_w2, conv0_1_t2,
        conv1_1_w1, conv1_1_t1, conv1_1_w2, conv1_1_t2,
        conv2_1_w1, conv2_1_t1, conv2_1_w2, conv2_1_t2,
        conv3_1_w1, conv3_1_t1, conv3_1_w2, conv3_1_t2,
        conv0_2_w1, conv0_2_t1, conv0_2_w2, conv0_2_t2,
        conv1_2_w1, conv1_2_t1, conv1_2_w2, conv1_2_t2,
        conv2_2_w1, conv2_2_t1, conv2_2_w2, conv2_2_t2,
        conv0_3_w1, conv0_3_t1, conv0_3_w2, conv0_3_t2,
        conv1_3_w1, conv1_3_t1, conv1_3_w2, conv1_3_t2,
        conv0_4_w1, conv0_4_t1, conv0_4_w2, conv0_4_t2,
        final_w, final_b,
    )


# speedup vs baseline: 1.2555x; 1.1439x over previous
import math

import jax
import jax.numpy as jnp
from jax.experimental import pallas as pl
from jax.experimental.pallas import tpu as pltpu


_BLOCK_NAMES = (
    "conv0_0", "conv1_0", "conv2_0", "conv3_0", "conv4_0",
    "conv0_1", "conv1_1", "conv2_1", "conv3_1",
    "conv0_2", "conv1_2", "conv2_2",
    "conv0_3", "conv1_3",
    "conv0_4",
)


def _lerp_coeffs(size_in):
    size_out = 2 * size_in
    if size_in == 1:
        return tuple((0, 0, 0.0) for _ in range(size_out))
    coeffs = []
    for o in range(size_out):
        src = o * (size_in - 1) / (size_out - 1)
        lo = min(int(math.floor(src)), size_in - 2)
        coeffs.append((lo, lo + 1, float(src - lo)))
    return tuple(coeffs)


def _upsample2x(v):
    n, h, w, c = v.shape
    x = v.astype(jnp.float32)

    def lerp_axis2(u, coeffs):
        pieces = []
        for lo, hi, f in coeffs:
            if f == 0.0:
                pieces.append(u[:, :, lo, :])
            elif f == 1.0:
                pieces.append(u[:, :, hi, :])
            else:
                pieces.append((1.0 - f) * u[:, :, lo, :] + f * u[:, :, hi, :])
        return jnp.stack(pieces, axis=1)

    y = lerp_axis2(x, _lerp_coeffs(w))
    z = lerp_axis2(y, _lerp_coeffs(h))
    return z.astype(jnp.bfloat16)


def _maxpool2x2(v):
    n, h, w, c = v.shape
    vr = v.reshape(n, h // 2, 2, w, c)
    m = jnp.maximum(vr[:, :, 0], vr[:, :, 1])
    pieces = [jnp.maximum(m[:, :, 2 * j, :], m[:, :, 2 * j + 1, :])
              for j in range(w // 2)]
    return jnp.stack(pieces, axis=2)


def _conv3x3_bn_relu(v, w_ref, t_ref):
    n, h, w, c = v.shape
    cout = w_ref.shape[-1]
    zh = jnp.zeros((n, 1, w, c), v.dtype)
    p = jnp.concatenate([zh, v, zh], axis=1)
    zw = jnp.zeros((n, h + 2, 1, c), v.dtype)
    p = jnp.concatenate([zw, p, zw], axis=2)

    if w % 8 == 0:
        z = jnp.concatenate(
            [p[:, :, 0:w, :], p[:, :, 1:w + 1, :], p[:, :, 2:w + 2, :]],
            axis=-1)
        z3 = z.reshape(n, (h + 2) * w, 3 * c)
        acc = None
        for dy in range(3):
            op = z3[:, dy * w:dy * w + h * w, :].reshape(n * h * w, 3 * c)
            d = jnp.dot(op, w_ref[dy * 3 * c:(dy + 1) * 3 * c, :],
                        preferred_element_type=jnp.float32)
            acc = d if acc is None else acc + d
    else:
        cols = []
        for dy in range(3):
            for dx in range(3):
                cols.append(
                    p[:, dy:dy + h, dx:dx + w, :].reshape(n * h * w, c))
        patches = jnp.concatenate(cols, axis=-1)
        acc = jnp.dot(patches, w_ref[...],
                      preferred_element_type=jnp.float32)
    y = jnp.maximum(acc + t_ref[...], 0.0)
    return y.astype(jnp.bfloat16).reshape(n, h, w, cout)


def _unet_kernel(*refs):
    x_ref = refs[0]
    o_ref = refs[-1]
    wrefs = refs[1:-1]
    blk = {name: wrefs[4 * i:4 * i + 4] for i, name in enumerate(_BLOCK_NAMES)}
    final_w, final_b = wrefs[60], wrefs[61]

    def block(inputs, name):
        w1, t1, w2, t2 = blk[name]
        v = inputs[0] if len(inputs) == 1 else jnp.concatenate(inputs, axis=-1)
        y1 = _conv3x3_bn_relu(v, w1, t1)
        return _conv3x3_bn_relu(y1, w2, t2)

    up = _upsample2x
    pool = _maxpool2x2

    x = x_ref[...]
    x0_0 = block([x], "conv0_0")
    x1_0 = block([pool(x0_0)], "conv1_0")
    x0_1 = block([x0_0, up(x1_0)], "conv0_1")

    x2_0 = block([pool(x1_0)], "conv2_0")
    x1_1 = block([x1_0, up(x2_0)], "conv1_1")
    x0_2 = block([x0_0, x0_1, up(x1_1)], "conv0_2")

    x3_0 = block([pool(x2_0)], "conv3_0")
    x2_1 = block([x2_0, up(x3_0)], "conv2_1")
    x1_2 = block([x1_0, x1_1, up(x2_1)], "conv1_2")
    x0_3 = block([x0_0, x0_1, x0_2, up(x1_2)], "conv0_3")

    x4_0 = block([pool(x3_0)], "conv4_0")
    x3_1 = block([x3_0, up(x4_0)], "conv3_1")
    x2_2 = block([x2_0, x2_1, up(x3_1)], "conv2_2")
    x1_3 = block([x1_0, x1_1, x1_2, up(x2_2)], "conv1_3")
    x0_4 = block([x0_0, x0_1, x0_2, x0_3, up(x1_3)], "conv0_4")

    n, h, w, c = x0_4.shape
    k = final_w.shape[-1]
    y = jnp.dot(x0_4.reshape(n * h * w, c), final_w[...],
                preferred_element_type=jnp.float32) + final_b[...]
    o_ref[...] = y.reshape(n, h, w, k)


def _full_spec(shape):
    ndim = len(shape)
    return pl.BlockSpec(tuple(shape), lambda i, _n=ndim: (0,) * _n)


@jax.jit
def _forward(x, *weights):
    n, hh, ww = x.shape[0], x.shape[2], x.shape[3]
    xh = jnp.transpose(x, (0, 2, 3, 1)).astype(jnp.bfloat16)
    num_classes = weights[-2].shape[-1]

    out = pl.pallas_call(
        _unet_kernel,
        out_shape=jax.ShapeDtypeStruct((n, hh, ww, num_classes), jnp.float32),
        grid=(1,),
        in_specs=[_full_spec(xh.shape)]
                 + [_full_spec(wt.shape) for wt in weights],
        out_specs=_full_spec((n, hh, ww, num_classes)),
        compiler_params=pltpu.CompilerParams(
            dimension_semantics=("arbitrary",),
            vmem_limit_bytes=100 * 1024 * 1024,
        ),
    )(xh, *weights)
    return jnp.transpose(out, (0, 3, 1, 2))


def kernel(x, conv0_0_w1, conv0_0_t1, conv0_0_w2, conv0_0_t2, conv1_0_w1, conv1_0_t1, conv1_0_w2, conv1_0_t2, conv2_0_w1, conv2_0_t1, conv2_0_w2, conv2_0_t2, conv3_0_w1, conv3_0_t1, conv3_0_w2, conv3_0_t2, conv4_0_w1, conv4_0_t1, conv4_0_w2, conv4_0_t2, conv0_1_w1, conv0_1_t1, conv0_1_w2, conv0_1_t2, conv1_1_w1, conv1_1_t1, conv1_1_w2, conv1_1_t2, conv2_1_w1, conv2_1_t1, conv2_1_w2, conv2_1_t2, conv3_1_w1, conv3_1_t1, conv3_1_w2, conv3_1_t2, conv0_2_w1, conv0_2_t1, conv0_2_w2, conv0_2_t2, conv1_2_w1, conv1_2_t1, conv1_2_w2, conv1_2_t2, conv2_2_w1, conv2_2_t1, conv2_2_w2, conv2_2_t2, conv0_3_w1, conv0_3_t1, conv0_3_w2, conv0_3_t2, conv1_3_w1, conv1_3_t1, conv1_3_w2, conv1_3_t2, conv0_4_w1, conv0_4_t1, conv0_4_w2, conv0_4_t2, final_w, final_b):
    return _forward(
        x,
        conv0_0_w1, conv0_0_t1, conv0_0_w2, conv0_0_t2,
        conv1_0_w1, conv1_0_t1, conv1_0_w2, conv1_0_t2,
        conv2_0_w1, conv2_0_t1, conv2_0_w2, conv2_0_t2,
        conv3_0_w1, conv3_0_t1, conv3_0_w2, conv3_0_t2,
        conv4_0_w1, conv4_0_t1, conv4_0_w2, conv4_0_t2,
        conv0_1_w1, conv0_1_t1, conv0_1_w2, conv0_1_t2,
        conv1_1_w1, conv1_1_t1, conv1_1_w2, conv1_1_t2,
        conv2_1_w1, conv2_1_t1, conv2_1_w2, conv2_1_t2,
        conv3_1_w1, conv3_1_t1, conv3_1_w2, conv3_1_t2,
        conv0_2_w1, conv0_2_t1, conv0_2_w2, conv0_2_t2,
        conv1_2_w1, conv1_2_t1, conv1_2_w2, conv1_2_t2,
        conv2_2_w1, conv2_2_t1, conv2_2_w2, conv2_2_t2,
        conv0_3_w1, conv0_3_t1, conv0_3_w2, conv0_3_t2,
        conv1_3_w1, conv1_3_t1, conv1_3_w2, conv1_3_t2,
        conv0_4_w1, conv0_4_t1, conv0_4_w2, conv0_4_t2,
        final_w, final_b,
    )
